# baseline (speedup 1.0000x reference)
import functools

import jax
import jax.numpy as jnp
from jax import lax
from jax.experimental import pallas as pl
from jax.experimental.pallas import tpu as pltpu
from jax.experimental.pallas import tpu_sc as plsc

try:
    _info = plsc.get_sparse_core_info()
    NUM_CORES, NUM_SUBCORES = int(_info.num_cores), int(_info.num_subcores)
except Exception:
    NUM_CORES, NUM_SUBCORES = 2, 16
NUM_WORKERS = NUM_CORES * NUM_SUBCORES

CHUNK_ROWS = 32
NBUF = 2


@functools.lru_cache(maxsize=None)
def _build(batch, num_classes):
    rows_per_worker = batch // NUM_WORKERS
    assert rows_per_worker * NUM_WORKERS == batch
    assert rows_per_worker % CHUNK_ROWS == 0
    n_chunks = rows_per_worker // CHUNK_ROWS

    mesh = plsc.VectorSubcoreMesh(core_axis_name="c", subcore_axis_name="s")

    @functools.partial(
        pl.kernel,
        mesh=mesh,
        out_type=jax.ShapeDtypeStruct((batch, num_classes), jnp.float32),
        compiler_params=pltpu.CompilerParams(
            skip_device_barrier=True, needs_layout_passes=False),
        scratch_types=[
            pltpu.VMEM((NBUF, CHUNK_ROWS, num_classes), jnp.float32),
            pltpu.VMEM((rows_per_worker,), jnp.int32),
        ]
        + [pltpu.SemaphoreType.DMA] * NBUF,
    )
    def onehot(idx_hbm, zeros_hbm, out_hbm, zbuf, idx_v, *sems):
        wid = lax.axis_index("s") * NUM_CORES + lax.axis_index("c")
        row0 = wid * rows_per_worker
        zeros16 = jnp.zeros((16,), jnp.float32)
        iota16 = lax.iota(jnp.int32, 16)

        def chunk_rows(c, b, make_vec):
            for g in range(CHUNK_ROWS // 16):
                v16 = idx_v[pl.ds(c * CHUNK_ROWS + g * 16, 16)]
                for j in range(16):
                    v = v16[j]
                    c0 = pl.multiple_of((v >> 4) << 4, 16)
                    zbuf[b, g * 16 + j, pl.ds(c0, 16)] = make_vec(v)

        pltpu.sync_copy(idx_hbm.at[pl.ds(row0, rows_per_worker)], idx_v)
        for b in range(NBUF):
            pltpu.sync_copy(zeros_hbm, zbuf.at[b])

        handles = [None] * NBUF
        for c in range(n_chunks):
            b = c % NBUF
            if handles[b] is not None:
                handles[b].wait()
                chunk_rows(c - NBUF, b, lambda v: zeros16)
            chunk_rows(
                c, b,
                lambda v: jnp.where(iota16 == (v & 15), 1.0, 0.0).astype(jnp.float32),
            )
            handles[b] = pltpu.async_copy(
                zbuf.at[b],
                out_hbm.at[pl.ds(row0 + c * CHUNK_ROWS, CHUNK_ROWS)],
                sems[b],
            )
        for h in handles:
            if h is not None:
                h.wait()

    return onehot


def kernel(indices, eye):
    batch = indices.shape[0]
    num_classes = eye.shape[0]
    zeros = jnp.zeros((CHUNK_ROWS, num_classes), jnp.float32)
    return _build(batch, num_classes)(indices, zeros)


# speedup vs baseline: 1.0030x; 1.0030x over previous
import functools

import jax
import jax.numpy as jnp
from jax import lax
from jax.experimental import pallas as pl
from jax.experimental.pallas import tpu as pltpu
from jax.experimental.pallas import tpu_sc as plsc

try:
    _info = plsc.get_sparse_core_info()
    NUM_CORES, NUM_SUBCORES = int(_info.num_cores), int(_info.num_subcores)
except Exception:
    NUM_CORES, NUM_SUBCORES = 2, 16
NUM_WORKERS = NUM_CORES * NUM_SUBCORES

CHUNK_ROWS = 32
NBUF = 2


@functools.lru_cache(maxsize=None)
def _build(batch, num_classes):
    rows_per_worker = batch // NUM_WORKERS
    assert rows_per_worker * NUM_WORKERS == batch
    assert rows_per_worker % CHUNK_ROWS == 0
    n_chunks = rows_per_worker // CHUNK_ROWS

    mesh = plsc.VectorSubcoreMesh(core_axis_name="c", subcore_axis_name="s")

    @functools.partial(
        pl.kernel,
        mesh=mesh,
        out_type=jax.ShapeDtypeStruct((batch, num_classes), jnp.float32),
        compiler_params=pltpu.CompilerParams(skip_device_barrier=True),
        scratch_types=[
            pltpu.VMEM((NBUF, CHUNK_ROWS, num_classes), jnp.float32),
            pltpu.VMEM((rows_per_worker,), jnp.int32),
        ]
        + [pltpu.SemaphoreType.DMA] * NBUF,
    )
    def onehot(idx_hbm, zeros_hbm, out_hbm, zbuf, idx_v, *sems):
        wid = lax.axis_index("s") * NUM_CORES + lax.axis_index("c")
        row0 = wid * rows_per_worker
        zeros16 = jnp.zeros((16,), jnp.float32)
        iota16 = lax.iota(jnp.int32, 16)

        def chunk_rows(c, b, make_vec):
            for g in range(CHUNK_ROWS // 16):
                v16 = idx_v[pl.ds(c * CHUNK_ROWS + g * 16, 16)]
                for j in range(16):
                    v = v16[j]
                    c0 = pl.multiple_of((v >> 4) << 4, 16)
                    zbuf[b, g * 16 + j, pl.ds(c0, 16)] = make_vec(v)

        pltpu.sync_copy(idx_hbm.at[pl.ds(row0, rows_per_worker)], idx_v)
        for b in range(NBUF):
            pltpu.sync_copy(zeros_hbm, zbuf.at[b])

        handles = [None] * NBUF
        for c in range(n_chunks):
            b = c % NBUF
            if handles[b] is not None:
                handles[b].wait()
                chunk_rows(c - NBUF, b, lambda v: zeros16)
            chunk_rows(
                c, b,
                lambda v: jnp.where(iota16 == (v & 15), 1.0, 0.0).astype(jnp.float32),
            )
            handles[b] = pltpu.async_copy(
                zbuf.at[b],
                out_hbm.at[pl.ds(row0 + c * CHUNK_ROWS, CHUNK_ROWS)],
                sems[b],
            )
        for h in handles:
            if h is not None:
                h.wait()

    return onehot


def kernel(indices, eye):
    batch = indices.shape[0]
    num_classes = eye.shape[0]
    zeros = jnp.zeros((CHUNK_ROWS, num_classes), jnp.float32)
    return _build(batch, num_classes)(indices, zeros)
